# baseline (speedup 1.0000x reference)
import functools

import jax
import jax.numpy as jnp
from jax import lax
from jax.experimental import pallas as pl
from jax.experimental.pallas import tpu as pltpu
from jax.experimental.pallas import tpu_sc as plsc

_NC = 2
_NS = 16
_NW = _NC * _NS
_L = 16
_NBUF = 4
_UNROLL = 8


def _winner_weights(labels, n, dtype):
    lab = labels.astype(jnp.int32)
    idx = jnp.arange(n, dtype=jnp.int32)
    later_dup = (lab[None, :] == lab[:, None]) & (idx[None, :] > idx[:, None])
    return (~later_dup.any(axis=1)).astype(dtype)


def _make_sc_kernel(n_planes, plane, chunk):
    mesh = plsc.VectorSubcoreMesh(core_axis_name="c", subcore_axis_name="s")

    @functools.partial(
        pl.kernel,
        mesh=mesh,
        out_type=jax.ShapeDtypeStruct((plane,), jnp.float32),
        scratch_types=[
            pltpu.VMEM((_NBUF, chunk), jnp.float32),
            pltpu.VMEM((chunk,), jnp.float32),
            pltpu.VMEM((n_planes, _L), jnp.float32),
        ]
        + [pltpu.SemaphoreType.DMA] * _NBUF
        + [pltpu.SemaphoreType.DMA],
    )
    def sc_kernel(masks_hbm, wexp_hbm, out_hbm, stage_v, acc_v, w_v, *sems):
        wsem = sems[_NBUF]
        wid = lax.axis_index("s") * _NC + lax.axis_index("c")
        base = wid * chunk

        pltpu.async_copy(wexp_hbm, w_v, wsem).wait()

        for b in range(_NBUF):
            pltpu.async_copy(
                masks_hbm.at[pl.ds(b * plane + base, chunk)],
                stage_v.at[b],
                sems[b],
            )

        def zero_body(j, _):
            acc_v[pl.ds(j * _L, _L)] = jnp.zeros((_L,), jnp.float32)
            return _

        lax.fori_loop(0, chunk // _L, zero_body, None)

        def group_body(g, _):
            for b in range(_NBUF):
                i = g * _NBUF + b
                pltpu.make_async_copy(
                    masks_hbm.at[pl.ds(base, chunk)], stage_v.at[b], sems[b]
                ).wait()
                wv = w_v[i]

                def add_body(j, _):
                    for u in range(_UNROLL):
                        sl = pl.ds((j * _UNROLL + u) * _L, _L)
                        acc_v[sl] = acc_v[sl] + wv * stage_v[b, sl]
                    return _

                lax.fori_loop(0, chunk // (_L * _UNROLL), add_body, None)

                @pl.when(i + _NBUF < n_planes)
                def _():
                    pltpu.async_copy(
                        masks_hbm.at[pl.ds((i + _NBUF) * plane + base, chunk)],
                        stage_v.at[b],
                        sems[b],
                    )
            return _

        lax.fori_loop(0, n_planes // _NBUF, group_body, None)

        pltpu.sync_copy(acc_v, out_hbm.at[pl.ds(base, chunk)])

    return sc_kernel


def kernel(inpt, masks, labels):
    n, h, w = masks.shape
    plane = h * w
    chunk = plane // _NW
    wts = _winner_weights(labels, n, masks.dtype)
    wexp = jnp.broadcast_to(wts[:, None], (n, _L))

    sc_kernel = _make_sc_kernel(n, plane, chunk)
    std_flat = sc_kernel(masks.reshape(n * plane), wexp)
    return (inpt, std_flat.reshape(h, w))


# speedup vs baseline: 5.9153x; 5.9153x over previous
import functools

import jax
import jax.numpy as jnp
from jax import lax
from jax.experimental import pallas as pl
from jax.experimental.pallas import tpu as pltpu
from jax.experimental.pallas import tpu_sc as plsc

_NC = 2
_NS = 16
_NW = _NC * _NS
_L = 16
_GRP = 4
_UNROLL = 8


def _winner_weights(labels, n, dtype):
    lab = labels.astype(jnp.int32)
    idx = jnp.arange(n, dtype=jnp.int32)
    later_dup = (lab[None, :] == lab[:, None]) & (idx[None, :] > idx[:, None])
    return (~later_dup.any(axis=1)).astype(dtype)


def _make_sc_kernel(n_planes, h, w):
    mesh = plsc.VectorSubcoreMesh(core_axis_name="c", subcore_axis_name="s")
    n_groups = n_planes // _GRP
    rows = h // _NW
    chunk = rows * w

    @functools.partial(
        pl.kernel,
        mesh=mesh,
        out_type=jax.ShapeDtypeStruct((h, w), jnp.float32),
        scratch_types=[
            pltpu.VMEM((2, _GRP, rows, w), jnp.float32),
            pltpu.VMEM((rows, w), jnp.float32),
            pltpu.VMEM((n_planes, _L), jnp.float32),
            pltpu.SemaphoreType.DMA,
            pltpu.SemaphoreType.DMA,
            pltpu.SemaphoreType.DMA,
        ],
        compiler_params=pltpu.CompilerParams(use_tc_tiling_on_sc=True),
    )
    def sc_kernel(masks_hbm, wexp_hbm, out_hbm, stage_v, acc_v, w_v, sem0, sem1, wsem):
        wid = lax.axis_index("s") * _NC + lax.axis_index("c")
        r0 = wid * rows
        gsems = (sem0, sem1)

        pltpu.async_copy(wexp_hbm, w_v, wsem).wait()

        def issue_group(g, gb):
            for p in range(_GRP):
                pltpu.async_copy(
                    masks_hbm.at[g * _GRP + p, pl.ds(r0, rows), :],
                    stage_v.at[gb, p],
                    gsems[gb],
                )

        def drain_group(gb):
            for p in range(_GRP):
                pltpu.make_async_copy(
                    masks_hbm.at[0, pl.ds(r0, rows), :], stage_v.at[gb, p], gsems[gb]
                ).wait()

        issue_group(0, 0)
        issue_group(1, 1)

        @plsc.parallel_loop(0, chunk // _L, unroll=_UNROLL)
        def _(j):
            r = j // (w // _L)
            c = (j % (w // _L)) * _L
            acc_v[r, pl.ds(c, _L)] = jnp.zeros((_L,), jnp.float32)

        def do_group(g, gb):
            drain_group(gb)
            w0 = w_v[g * _GRP]
            w1 = w_v[g * _GRP + 1]
            w2 = w_v[g * _GRP + 2]
            w3 = w_v[g * _GRP + 3]

            @plsc.parallel_loop(0, chunk // _L, unroll=_UNROLL)
            def _(j):
                r = j // (w // _L)
                c = (j % (w // _L)) * _L
                sl = pl.ds(c, _L)
                acc_v[r, sl] = (
                    acc_v[r, sl]
                    + (w0 * stage_v[gb, 0, r, sl] + w1 * stage_v[gb, 1, r, sl])
                    + (w2 * stage_v[gb, 2, r, sl] + w3 * stage_v[gb, 3, r, sl])
                )

            @pl.when(g + 2 < n_groups)
            def _():
                issue_group(g + 2, gb)

        def pair_body(p, _):
            do_group(p * 2, 0)
            do_group(p * 2 + 1, 1)
            return 0

        lax.fori_loop(0, n_groups // 2, pair_body, 0)
        if n_groups % 2:
            do_group(n_groups - 1, 0)

        pltpu.sync_copy(acc_v, out_hbm.at[pl.ds(r0, rows), :])

    return sc_kernel


def kernel(inpt, masks, labels):
    n, h, w = masks.shape
    wts = _winner_weights(labels, n, masks.dtype)
    wexp = jnp.broadcast_to(wts[:, None], (n, _L))

    sc_kernel = _make_sc_kernel(n, h, w)
    std_mask = sc_kernel(masks, wexp)
    return (inpt, std_mask)


# speedup vs baseline: 6.2607x; 1.0584x over previous
import functools

import jax
import jax.numpy as jnp
from jax import lax
from jax.experimental import pallas as pl
from jax.experimental.pallas import tpu as pltpu
from jax.experimental.pallas import tpu_sc as plsc

_NC = 2
_NS = 16
_NW = _NC * _NS
_L = 16
_GRP = 4
_PAD = 2 * _GRP
_UNROLL = 8


def _make_sc_kernel(n_planes, h, w):
    mesh = plsc.VectorSubcoreMesh(core_axis_name="c", subcore_axis_name="s")
    n_idx = n_planes + _PAD - 1
    rows = h // _NW
    chunk = rows * w

    @functools.partial(
        pl.kernel,
        mesh=mesh,
        out_type=jax.ShapeDtypeStruct((h, w), jnp.float32),
        scratch_types=[
            pltpu.VMEM((2, _GRP, rows, w), jnp.float32),
            pltpu.VMEM((rows, w), jnp.float32),
            pltpu.VMEM((n_idx, _L), jnp.float32),
            pltpu.VMEM((n_idx, _L), jnp.int32),
            pltpu.VMEM((_L,), jnp.int32),
            pltpu.SemaphoreType.DMA,
            pltpu.SemaphoreType.DMA,
            pltpu.SemaphoreType.DMA,
        ],
        compiler_params=pltpu.CompilerParams(
            use_tc_tiling_on_sc=True, needs_layout_passes=False
        ),
    )
    def sc_kernel(
        masks_hbm, wexp_hbm, ordexp_hbm, meta_hbm, out_hbm,
        stage_v, acc_v, w_v, ord_v, meta_v, sem0, sem1, wsem,
    ):
        wid = lax.axis_index("s") * _NC + lax.axis_index("c")
        r0 = wid * rows
        gsems = (sem0, sem1)

        pltpu.async_copy(wexp_hbm, w_v, wsem)
        pltpu.async_copy(ordexp_hbm, ord_v, wsem)
        pltpu.make_async_copy(wexp_hbm, w_v, wsem).wait()
        pltpu.make_async_copy(ordexp_hbm, ord_v, wsem).wait()
        pltpu.async_copy(meta_hbm, meta_v, wsem).wait()
        npairs = jnp.max(meta_v[...])
        ng = npairs * 2

        def issue_group(g, gb):
            for p in range(_GRP):
                idx = jnp.max(ord_v[g * _GRP + p])
                pltpu.async_copy(
                    masks_hbm.at[idx, pl.ds(r0, rows), :],
                    stage_v.at[gb, p],
                    gsems[gb],
                )

        def drain_group(gb):
            for p in range(_GRP):
                pltpu.make_async_copy(
                    masks_hbm.at[0, pl.ds(r0, rows), :], stage_v.at[gb, p], gsems[gb]
                ).wait()

        issue_group(0, 0)
        issue_group(1, 1)

        @plsc.parallel_loop(0, chunk // _L, unroll=_UNROLL)
        def _(j):
            r = j // (w // _L)
            c = (j % (w // _L)) * _L
            acc_v[r, pl.ds(c, _L)] = jnp.zeros((_L,), jnp.float32)

        def do_group(g, gb):
            drain_group(gb)
            w0 = w_v[g * _GRP]
            w1 = w_v[g * _GRP + 1]
            w2 = w_v[g * _GRP + 2]
            w3 = w_v[g * _GRP + 3]

            @plsc.parallel_loop(0, chunk // _L, unroll=_UNROLL)
            def _(j):
                r = j // (w // _L)
                c = (j % (w // _L)) * _L
                sl = pl.ds(c, _L)
                acc_v[r, sl] = (
                    acc_v[r, sl]
                    + (w0 * stage_v[gb, 0, r, sl] + w1 * stage_v[gb, 1, r, sl])
                    + (w2 * stage_v[gb, 2, r, sl] + w3 * stage_v[gb, 3, r, sl])
                )

            @pl.when(g + 2 < ng)
            def _():
                issue_group(g + 2, gb)

        def pair_body(p, _):
            do_group(p * 2, 0)
            do_group(p * 2 + 1, 1)
            return 0

        lax.fori_loop(0, npairs, pair_body, 0)

        pltpu.sync_copy(acc_v, out_hbm.at[pl.ds(r0, rows), :])

    return sc_kernel


def kernel(inpt, masks, labels):
    n, h, w = masks.shape
    n_idx = n + _PAD - 1

    lab = labels.astype(jnp.int32)
    iota = jnp.arange(n, dtype=jnp.int32)
    later_dup = (lab[None, :] == lab[:, None]) & (iota[None, :] > iota[:, None])
    keep = ~later_dup.any(axis=1)
    k = jnp.sum(keep.astype(jnp.int32))
    npairs = (k + _PAD - 1) // _PAD

    order = jnp.argsort(~keep, stable=True).astype(jnp.int32)
    order_pad = jnp.concatenate([order, jnp.zeros((_PAD - 1,), jnp.int32)])
    wsort = keep[order].astype(masks.dtype)
    wsort_pad = jnp.concatenate([wsort, jnp.zeros((_PAD - 1,), masks.dtype)])

    wexp = jnp.broadcast_to(wsort_pad[:, None], (n_idx, _L))
    ordexp = jnp.broadcast_to(order_pad[:, None], (n_idx, _L))
    meta = jnp.full((_L,), npairs, jnp.int32)

    sc_kernel = _make_sc_kernel(n, h, w)
    std_mask = sc_kernel(masks, wexp, ordexp, meta)
    return (inpt, std_mask)


# speedup vs baseline: 6.4122x; 1.0242x over previous
import functools

import jax
import jax.numpy as jnp
from jax import lax
from jax.experimental import pallas as pl
from jax.experimental.pallas import tpu as pltpu
from jax.experimental.pallas import tpu_sc as plsc

_NC = 2
_NS = 16
_NW = _NC * _NS
_L = 16
_GRP = 4
_PAD = 2 * _GRP
_UNROLL = 8


def _make_sc_kernel(n_planes, h, w):
    mesh = plsc.VectorSubcoreMesh(core_axis_name="c", subcore_axis_name="s")
    n_idx = n_planes + _PAD - 1
    rows = h // _NW
    chunk = rows * w

    @functools.partial(
        pl.kernel,
        mesh=mesh,
        out_type=jax.ShapeDtypeStruct((h, w), jnp.float32),
        scratch_types=[
            pltpu.VMEM((2, _GRP, rows, w), jnp.float32),
            pltpu.VMEM((rows, w), jnp.float32),
            pltpu.VMEM((n_idx, _L), jnp.float32),
            pltpu.VMEM((n_idx, _L), jnp.int32),
            pltpu.VMEM((_L,), jnp.int32),
            pltpu.SemaphoreType.DMA,
            pltpu.SemaphoreType.DMA,
            pltpu.SemaphoreType.DMA,
        ],
        compiler_params=pltpu.CompilerParams(
            use_tc_tiling_on_sc=True, needs_layout_passes=False
        ),
    )
    def sc_kernel(
        masks_hbm, wexp_hbm, ordexp_hbm, meta_hbm, out_hbm,
        stage_v, acc_v, w_v, ord_v, meta_v, sem0, sem1, wsem,
    ):
        wid = lax.axis_index("s") * _NC + lax.axis_index("c")
        r0 = wid * rows
        gsems = (sem0, sem1)

        pltpu.async_copy(wexp_hbm, w_v, wsem)
        pltpu.async_copy(ordexp_hbm, ord_v, wsem)
        pltpu.make_async_copy(wexp_hbm, w_v, wsem).wait()
        pltpu.make_async_copy(ordexp_hbm, ord_v, wsem).wait()
        pltpu.async_copy(meta_hbm, meta_v, wsem).wait()
        npairs = jnp.max(meta_v[...])
        ng = npairs * 2

        def issue_group(g, gb):
            for p in range(_GRP):
                idx = jnp.max(ord_v[g * _GRP + p])
                pltpu.async_copy(
                    masks_hbm.at[idx, pl.ds(r0, rows), :],
                    stage_v.at[gb, p],
                    gsems[gb],
                )

        def drain_group(gb):
            for p in range(_GRP):
                pltpu.make_async_copy(
                    masks_hbm.at[0, pl.ds(r0, rows), :], stage_v.at[gb, p], gsems[gb]
                ).wait()

        issue_group(0, 0)
        issue_group(1, 1)

        @plsc.parallel_loop(0, chunk // _L, unroll=_UNROLL)
        def _(j):
            r = j // (w // _L)
            c = (j % (w // _L)) * _L
            acc_v[r, pl.ds(c, _L)] = jnp.zeros((_L,), jnp.float32)

        def do_group(g, gb):
            drain_group(gb)
            w0 = w_v[g * _GRP]
            w1 = w_v[g * _GRP + 1]
            w2 = w_v[g * _GRP + 2]
            w3 = w_v[g * _GRP + 3]

            @plsc.parallel_loop(0, chunk // _L, unroll=_UNROLL)
            def _(j):
                r = j // (w // _L)
                c = (j % (w // _L)) * _L
                sl = pl.ds(c, _L)
                acc_v[r, sl] = (
                    acc_v[r, sl]
                    + (w0 * stage_v[gb, 0, r, sl] + w1 * stage_v[gb, 1, r, sl])
                    + (w2 * stage_v[gb, 2, r, sl] + w3 * stage_v[gb, 3, r, sl])
                )

            @pl.when(g + 2 < ng)
            def _():
                issue_group(g + 2, gb)

        def pair_body(p, _):
            do_group(p * 2, 0)
            do_group(p * 2 + 1, 1)
            return 0

        lax.fori_loop(0, npairs, pair_body, 0)

        pltpu.sync_copy(acc_v, out_hbm.at[pl.ds(r0, rows), :])

    return sc_kernel


def kernel(inpt, masks, labels):
    n, h, w = masks.shape
    n_idx = n + _PAD - 1

    lab = labels.astype(jnp.int32)
    iota = jnp.arange(n, dtype=jnp.int32)
    later_dup = (lab[None, :] == lab[:, None]) & (iota[None, :] > iota[:, None])
    keep = ~later_dup.any(axis=1)
    k = jnp.sum(keep.astype(jnp.int32))
    npairs = (k + _PAD - 1) // _PAD

    pos = jnp.cumsum(keep.astype(jnp.int32)) - 1
    slot = jnp.where(keep, pos, n_idx - 1)
    order_pad = jnp.zeros((n_idx,), jnp.int32).at[slot].set(iota)
    wsort_pad = (jnp.arange(n_idx, dtype=jnp.int32) < k).astype(masks.dtype)

    wexp = jnp.broadcast_to(wsort_pad[:, None], (n_idx, _L))
    ordexp = jnp.broadcast_to(order_pad[:, None], (n_idx, _L))
    meta = jnp.full((_L,), npairs, jnp.int32)

    sc_kernel = _make_sc_kernel(n, h, w)
    std_mask = sc_kernel(masks, wexp, ordexp, meta)
    return (inpt, std_mask)


# speedup vs baseline: 6.7543x; 1.0533x over previous
import functools

import jax
import jax.numpy as jnp
from jax import lax
from jax.experimental import pallas as pl
from jax.experimental.pallas import tpu as pltpu
from jax.experimental.pallas import tpu_sc as plsc

_NC = 2
_NS = 16
_NW = _NC * _NS
_L = 16
_GRP = 4
_PAD = 2 * _GRP
_UNROLL = 8


def _make_sc_kernel(n_planes, h, w):
    mesh = plsc.VectorSubcoreMesh(core_axis_name="c", subcore_axis_name="s")
    n_idx = n_planes + _PAD - 1
    rows = h // _NW
    chunk = rows * w

    @functools.partial(
        pl.kernel,
        mesh=mesh,
        out_type=jax.ShapeDtypeStruct((h, w), jnp.float32),
        scratch_types=[
            pltpu.VMEM((2, _GRP, rows, w), jnp.float32),
            pltpu.VMEM((rows, w), jnp.float32),
            pltpu.VMEM((n_idx, _L), jnp.float32),
            pltpu.VMEM((n_idx, _L), jnp.int32),
            pltpu.VMEM((_L,), jnp.int32),
            pltpu.SemaphoreType.DMA,
            pltpu.SemaphoreType.DMA,
            pltpu.SemaphoreType.DMA,
        ],
        compiler_params=pltpu.CompilerParams(
            use_tc_tiling_on_sc=True, needs_layout_passes=False
        ),
    )
    def sc_kernel(
        masks_hbm, wexp_hbm, ordexp_hbm, meta_hbm, out_hbm,
        stage_v, acc_v, w_v, ord_v, meta_v, sem0, sem1, wsem,
    ):
        wid = lax.axis_index("s") * _NC + lax.axis_index("c")
        r0 = wid * rows
        gsems = (sem0, sem1)

        pltpu.async_copy(wexp_hbm, w_v, wsem)
        pltpu.async_copy(ordexp_hbm, ord_v, wsem)
        pltpu.make_async_copy(wexp_hbm, w_v, wsem).wait()
        pltpu.make_async_copy(ordexp_hbm, ord_v, wsem).wait()
        pltpu.async_copy(meta_hbm, meta_v, wsem).wait()
        npairs = jnp.max(meta_v[...])
        ng = npairs * 2

        def issue_group(g, gb):
            for p in range(_GRP):
                idx = jnp.max(ord_v[g * _GRP + p])
                pltpu.async_copy(
                    masks_hbm.at[idx, pl.ds(r0, rows), :],
                    stage_v.at[gb, p],
                    gsems[gb],
                )

        def drain_group(gb):
            for p in range(_GRP):
                pltpu.make_async_copy(
                    masks_hbm.at[0, pl.ds(r0, rows), :], stage_v.at[gb, p], gsems[gb]
                ).wait()

        issue_group(0, 0)
        issue_group(1, 1)

        @plsc.parallel_loop(0, chunk // _L, unroll=_UNROLL)
        def _(j):
            r = j // (w // _L)
            c = (j % (w // _L)) * _L
            acc_v[r, pl.ds(c, _L)] = jnp.zeros((_L,), jnp.float32)

        def do_group(g, gb):
            drain_group(gb)
            w0 = w_v[g * _GRP]
            w1 = w_v[g * _GRP + 1]
            w2 = w_v[g * _GRP + 2]
            w3 = w_v[g * _GRP + 3]

            @plsc.parallel_loop(0, chunk // _L, unroll=_UNROLL)
            def _(j):
                r = j // (w // _L)
                c = (j % (w // _L)) * _L
                sl = pl.ds(c, _L)
                acc_v[r, sl] = (
                    acc_v[r, sl]
                    + (w0 * stage_v[gb, 0, r, sl] + w1 * stage_v[gb, 1, r, sl])
                    + (w2 * stage_v[gb, 2, r, sl] + w3 * stage_v[gb, 3, r, sl])
                )

            @pl.when(g + 2 < ng)
            def _():
                issue_group(g + 2, gb)

        def pair_body(p, _):
            do_group(p * 2, 0)
            do_group(p * 2 + 1, 1)
            return 0

        lax.fori_loop(0, npairs, pair_body, 0)

        pltpu.sync_copy(acc_v, out_hbm.at[pl.ds(r0, rows), :])

    return sc_kernel


_TC_GRID = 40


def _make_tc_kernel(n, h, w):
    def body(idx_ref, ktc_ref, m_ref, o_ref):
        i = pl.program_id(0)

        @pl.when(i == 0)
        def _():
            o_ref[...] = jnp.zeros_like(o_ref)

        wt = (i < ktc_ref[0]).astype(o_ref.dtype)
        o_ref[...] += wt * m_ref[0]

    grid_spec = pltpu.PrefetchScalarGridSpec(
        num_scalar_prefetch=2,
        grid=(_TC_GRID,),
        in_specs=[
            pl.BlockSpec((1, h, w), lambda i, idx_ref, ktc_ref: (idx_ref[i], 0, 0)),
        ],
        out_specs=pl.BlockSpec((h, w), lambda i, idx_ref, ktc_ref: (0, 0)),
    )
    return pl.pallas_call(
        body,
        grid_spec=grid_spec,
        out_shape=jax.ShapeDtypeStruct((h, w), jnp.float32),
    )


def kernel(inpt, masks, labels):
    n, h, w = masks.shape
    n_idx = n + _PAD - 1

    lab = labels.astype(jnp.int32)
    iota = jnp.arange(n, dtype=jnp.int32)
    later_dup = (lab[None, :] == lab[:, None]) & (iota[None, :] > iota[:, None])
    keep = ~later_dup.any(axis=1)
    k = jnp.sum(keep.astype(jnp.int32))

    ksc = jnp.maximum(((k * 6 // 10) + _PAD - 1) // _PAD * _PAD, _PAD)
    ktc = jnp.maximum(k - ksc, 0)
    npairs = ksc // _PAD

    pos = jnp.cumsum(keep.astype(jnp.int32)) - 1
    slot = jnp.where(keep, pos, n_idx - 1)
    order_pad = jnp.zeros((n_idx,), jnp.int32).at[slot].set(iota)
    wsort_pad = (jnp.arange(n_idx, dtype=jnp.int32) < k).astype(masks.dtype)

    wexp = jnp.broadcast_to(wsort_pad[:, None], (n_idx, _L))
    ordexp = jnp.broadcast_to(order_pad[:, None], (n_idx, _L))
    meta = jnp.full((_L,), npairs, jnp.int32)

    j = jnp.arange(_TC_GRID, dtype=jnp.int32)
    tc_idx = order_pad[ksc + jnp.clip(j, 0, jnp.maximum(ktc - 1, 0))]
    ktc_arr = jnp.full((1,), ktc, jnp.int32)

    sc_kernel = _make_sc_kernel(n, h, w)
    tc_kernel = _make_tc_kernel(n, h, w)
    sc_part = sc_kernel(masks, wexp, ordexp, meta)
    tc_part = tc_kernel(tc_idx, ktc_arr, masks)
    return (inpt, sc_part + tc_part)


# speedup vs baseline: 7.3219x; 1.0840x over previous
import functools

import jax
import jax.numpy as jnp
from jax import lax
from jax.experimental import pallas as pl
from jax.experimental.pallas import tpu as pltpu
from jax.experimental.pallas import tpu_sc as plsc

_NC = 2
_NS = 16
_NW = _NC * _NS
_L = 16
_GRP = 4
_PAD = 2 * _GRP
_UNROLL = 8


def _make_sc_kernel(n_planes, h, w):
    mesh = plsc.VectorSubcoreMesh(core_axis_name="c", subcore_axis_name="s")
    n_idx = n_planes + _PAD - 1
    rows = h // _NW
    chunk = rows * w

    @functools.partial(
        pl.kernel,
        mesh=mesh,
        out_type=jax.ShapeDtypeStruct((h, w), jnp.float32),
        scratch_types=[
            pltpu.VMEM((2, _GRP, rows, w), jnp.float32),
            pltpu.VMEM((rows, w), jnp.float32),
            pltpu.VMEM((n_idx, _L), jnp.int32),
            pltpu.VMEM((2, _L), jnp.int32),
            pltpu.SemaphoreType.DMA,
            pltpu.SemaphoreType.DMA,
            pltpu.SemaphoreType.DMA,
        ],
        compiler_params=pltpu.CompilerParams(
            use_tc_tiling_on_sc=True, needs_layout_passes=False
        ),
    )
    def sc_kernel(
        masks_hbm, ordexp_hbm, meta_hbm, out_hbm,
        stage_v, acc_v, ord_v, meta_v, sem0, sem1, wsem,
    ):
        wid = lax.axis_index("s") * _NC + lax.axis_index("c")
        r0 = wid * rows
        gsems = (sem0, sem1)

        pltpu.async_copy(ordexp_hbm, ord_v, wsem)
        pltpu.make_async_copy(ordexp_hbm, ord_v, wsem).wait()
        pltpu.async_copy(meta_hbm, meta_v, wsem).wait()
        npairs = jnp.max(meta_v[0])
        k = jnp.max(meta_v[1])
        ng = npairs * 2

        def issue_group(g, gb):
            for p in range(_GRP):
                idx = jnp.max(ord_v[g * _GRP + p])
                pltpu.async_copy(
                    masks_hbm.at[idx, pl.ds(r0, rows), :],
                    stage_v.at[gb, p],
                    gsems[gb],
                )

        def drain_group(gb):
            for p in range(_GRP):
                pltpu.make_async_copy(
                    masks_hbm.at[0, pl.ds(r0, rows), :], stage_v.at[gb, p], gsems[gb]
                ).wait()

        issue_group(0, 0)
        issue_group(1, 1)

        @plsc.parallel_loop(0, chunk // _L, unroll=_UNROLL)
        def _(j):
            r = j // (w // _L)
            c = (j % (w // _L)) * _L
            acc_v[r, pl.ds(c, _L)] = jnp.zeros((_L,), jnp.float32)

        def do_group(g, gb):
            drain_group(gb)
            ws = [
                jnp.broadcast_to(
                    jnp.where(g * _GRP + p < k, jnp.float32(1.0), jnp.float32(0.0)),
                    (_L,),
                )
                for p in range(_GRP)
            ]
            w0, w1, w2, w3 = ws

            @plsc.parallel_loop(0, chunk // _L, unroll=_UNROLL)
            def _(j):
                r = j // (w // _L)
                c = (j % (w // _L)) * _L
                sl = pl.ds(c, _L)
                acc_v[r, sl] = (
                    acc_v[r, sl]
                    + (w0 * stage_v[gb, 0, r, sl] + w1 * stage_v[gb, 1, r, sl])
                    + (w2 * stage_v[gb, 2, r, sl] + w3 * stage_v[gb, 3, r, sl])
                )

            @pl.when(g + 2 < ng)
            def _():
                issue_group(g + 2, gb)

        def pair_body(p, _):
            do_group(p * 2, 0)
            do_group(p * 2 + 1, 1)
            return 0

        lax.fori_loop(0, npairs, pair_body, 0)

        pltpu.sync_copy(acc_v, out_hbm.at[pl.ds(r0, rows), :])

    return sc_kernel


_TC_GRID = 40


def _make_tc_kernel(n, h, w):
    def body(idx_ref, ktc_ref, m_ref, o_ref):
        i = pl.program_id(0)

        @pl.when(i == 0)
        def _():
            o_ref[...] = jnp.zeros_like(o_ref)

        wt = (i < ktc_ref[0]).astype(o_ref.dtype)
        o_ref[...] += wt * m_ref[0]

    grid_spec = pltpu.PrefetchScalarGridSpec(
        num_scalar_prefetch=2,
        grid=(_TC_GRID,),
        in_specs=[
            pl.BlockSpec((1, h, w), lambda i, idx_ref, ktc_ref: (idx_ref[i], 0, 0)),
        ],
        out_specs=pl.BlockSpec((h, w), lambda i, idx_ref, ktc_ref: (0, 0)),
    )
    return pl.pallas_call(
        body,
        grid_spec=grid_spec,
        out_shape=jax.ShapeDtypeStruct((h, w), jnp.float32),
    )


def kernel(inpt, masks, labels):
    n, h, w = masks.shape
    n_idx = n + _PAD - 1

    lab = labels.astype(jnp.int32)
    iota = jnp.arange(n, dtype=jnp.int32)
    later_dup = (lab[None, :] == lab[:, None]) & (iota[None, :] > iota[:, None])
    keep = ~later_dup.any(axis=1)
    k = jnp.sum(keep.astype(jnp.int32))

    ksc = jnp.maximum(((k * 56 // 100) + _PAD - 1) // _PAD * _PAD, _PAD)
    ktc = jnp.maximum(k - ksc, 0)
    npairs = ksc // _PAD

    pos = jnp.cumsum(keep.astype(jnp.int32)) - 1
    slot = jnp.where(keep, pos, n_idx - 1)
    order_pad = jnp.zeros((n_idx,), jnp.int32).at[slot].set(iota)

    ordexp = jnp.broadcast_to(order_pad[:, None], (n_idx, _L))
    meta = jnp.stack([jnp.full((_L,), npairs, jnp.int32), jnp.full((_L,), k, jnp.int32)])

    j = jnp.arange(_TC_GRID, dtype=jnp.int32)
    tc_idx = order_pad[ksc + jnp.clip(j, 0, jnp.maximum(ktc - 1, 0))]
    ktc_arr = jnp.full((1,), ktc, jnp.int32)

    sc_kernel = _make_sc_kernel(n, h, w)
    tc_kernel = _make_tc_kernel(n, h, w)
    sc_part = sc_kernel(masks, ordexp, meta)
    tc_part = tc_kernel(tc_idx, ktc_arr, masks)
    return (inpt, sc_part + tc_part)
